# baseline (speedup 1.0000x reference)
import functools

import jax
import jax.numpy as jnp
from jax import lax
from jax.experimental import pallas as pl
from jax.experimental.pallas import tpu as pltpu
from jax.experimental.pallas import tpu_sc as plsc

NUM_CORES = 2
NUM_SUBCORES = 16
B_CORE0 = 576
B_CORE1 = 448


def _build(B, V, D, b0, b1):
    mesh = plsc.VectorSubcoreMesh(core_axis_name="c", subcore_axis_name="s")
    bmax = max(b0, b1)

    @functools.partial(
        pl.kernel,
        mesh=mesh,
        out_type=jax.ShapeDtypeStruct((B, D), jnp.float32),
        scratch_types=[
            pltpu.VMEM((bmax,), jnp.int32),
            pltpu.VMEM((bmax, D), jnp.float32),
            pltpu.SemaphoreType.DMA,
        ],
    )
    def emb(idx_hbm, table_hbm, out_hbm, idx_v, rows_v, sem):
        cid = lax.axis_index("c")
        sid = lax.axis_index("s")
        for my_cid, bw, core_base in ((0, b0, 0), (1, b1, NUM_SUBCORES * b0)):
            @pl.when(cid == my_cid)
            def _():
                base = core_base + sid * bw
                pltpu.sync_copy(idx_hbm.at[pl.ds(base, bw)], idx_v.at[pl.ds(0, bw)])
                pltpu.async_copy(
                    table_hbm.at[idx_v.at[pl.ds(0, bw)]],
                    rows_v.at[pl.ds(0, bw)],
                    sem,
                ).wait()
                pltpu.sync_copy(rows_v.at[pl.ds(0, bw)], out_hbm.at[pl.ds(base, bw)])

    return emb


def kernel(t, table):
    (B,) = t.shape
    V, D = table.shape
    emb = _build(B, V, D, B_CORE0, B_CORE1)
    return emb(t.astype(jnp.int32), table)


# speedup vs baseline: 1.0582x; 1.0582x over previous
import functools

import jax
import jax.numpy as jnp
from jax import lax
from jax.experimental import pallas as pl
from jax.experimental.pallas import tpu as pltpu
from jax.experimental.pallas import tpu_sc as plsc

NUM_CORES = 2
NUM_SUBCORES = 16
NW = NUM_CORES * NUM_SUBCORES


def _build(B, V, D):
    b_per_w = B // NW
    mesh = plsc.VectorSubcoreMesh(core_axis_name="c", subcore_axis_name="s")

    @functools.partial(
        pl.kernel,
        mesh=mesh,
        out_type=jax.ShapeDtypeStruct((B, D), jnp.float32),
        scratch_types=[
            pltpu.VMEM((b_per_w,), jnp.int32),
            pltpu.VMEM((b_per_w, D), jnp.float32),
            pltpu.SemaphoreType.DMA,
        ],
    )
    def emb(idx_hbm, table_hbm, out_hbm, idx_v, rows_v, sem):
        wid = lax.axis_index("s") * NUM_CORES + lax.axis_index("c")
        base = wid * b_per_w
        pltpu.sync_copy(idx_hbm.at[pl.ds(base, b_per_w)], idx_v)
        pltpu.async_copy(table_hbm.at[idx_v], rows_v, sem).wait()
        pltpu.sync_copy(rows_v, out_hbm.at[pl.ds(base, b_per_w)])

    return emb


def kernel(t, table):
    (B,) = t.shape
    V, D = table.shape
    emb = _build(B, V, D)
    return emb(t.astype(jnp.int32), table)
